# baseline (speedup 1.0000x reference)
import functools

import jax
import jax.numpy as jnp
from jax.experimental import pallas as pl
from jax.experimental.pallas import tpu as pltpu

N_NODES = 10000
BM = 400


_SELU_ALPHA = 1.6732632423543772848170429916717
_SELU_SCALE = 1.0507009873554804934193349852946


def _selu(x):
    neg = _SELU_ALPHA * (jnp.exp(jnp.minimum(x, 0.0)) - 1.0)
    return _SELU_SCALE * jnp.where(x > 0, x, neg)


def _s1_body(x_ref, w1_ref, o_ref):
    o_ref[...] = jnp.dot(x_ref[...], w1_ref[...],
                         preferred_element_type=jnp.float32)


def _pass1_body(adj_ref, s1_ref, b1_ref, w2_ref, s2_ref):
    h = jnp.dot(adj_ref[...], s1_ref[...],
                preferred_element_type=jnp.float32) + b1_ref[...]
    h = _selu(h)
    s2_ref[...] = jnp.dot(h, w2_ref[...], preferred_element_type=jnp.float32)


def _pass2_body(adj_ref, s2_ref, b2_ref, out_ref, acc_ref):
    i = pl.program_id(0)
    h = _selu(jnp.dot(adj_ref[...], s2_ref[...],
                            preferred_element_type=jnp.float32) + b2_ref[...])
    part = jnp.sum(h, axis=0, keepdims=True)

    @pl.when(i == 0)
    def _init():
        acc_ref[...] = part

    @pl.when(i > 0)
    def _acc():
        acc_ref[...] += part

    @pl.when(i == pl.num_programs(0) - 1)
    def _fin():
        p = _selu(acc_ref[...] * (1.0 / N_NODES))
        out_ref[...] = jax.nn.log_softmax(p, axis=1)


@jax.jit
def kernel(x, adj, W1, b1, W2, b2):
    n, f_in = x.shape
    h_dim = W1.shape[1]
    c_dim = W2.shape[1]
    b1r = b1.reshape(1, h_dim)
    b2r = b2.reshape(1, c_dim)

    s1 = pl.pallas_call(
        _s1_body,
        out_shape=jax.ShapeDtypeStruct((n, h_dim), jnp.float32),
    )(x, W1)

    num_blocks = n // BM
    s2 = pl.pallas_call(
        _pass1_body,
        grid=(num_blocks,),
        in_specs=[
            pl.BlockSpec((BM, n), lambda i: (i, 0)),
            pl.BlockSpec((n, h_dim), lambda i: (0, 0)),
            pl.BlockSpec((1, h_dim), lambda i: (0, 0)),
            pl.BlockSpec((h_dim, c_dim), lambda i: (0, 0)),
        ],
        out_specs=pl.BlockSpec((BM, c_dim), lambda i: (i, 0)),
        out_shape=jax.ShapeDtypeStruct((n, c_dim), jnp.float32),
    )(adj, s1, b1r, W2)

    out = pl.pallas_call(
        _pass2_body,
        grid=(num_blocks,),
        in_specs=[
            pl.BlockSpec((BM, n), lambda i: (i, 0)),
            pl.BlockSpec((n, c_dim), lambda i: (0, 0)),
            pl.BlockSpec((1, c_dim), lambda i: (0, 0)),
        ],
        out_specs=pl.BlockSpec((1, c_dim), lambda i: (0, 0)),
        out_shape=jax.ShapeDtypeStruct((1, c_dim), jnp.float32),
        scratch_shapes=[pltpu.VMEM((1, c_dim), jnp.float32)],
    )(adj, s2, b2r)

    return out
